# Initial kernel scaffold; baseline (speedup 1.0000x reference)
import jax
import jax.numpy as jnp
from jax.experimental import pallas as pl


def kernel(word_inputs, word_seq_lengths, char_inputs, char_seq_lengths, char_seq_recover, W):
    raise NotImplementedError("write your pallas kernel here")


# speedup vs baseline: 1.1587x; 1.1587x over previous
import functools

import jax
import jax.numpy as jnp
from jax import lax
from jax.experimental import pallas as pl
from jax.experimental.pallas import tpu as pltpu
from jax.experimental.pallas import tpu_sc as plsc

_NUM_CORES = 2
_NUM_SUBCORES = 16
_NUM_WORKERS = _NUM_CORES * _NUM_SUBCORES


@functools.lru_cache(maxsize=None)
def _make_sc_gather(V, D, B):
  assert B % _NUM_WORKERS == 0
  b_per_w = B // _NUM_WORKERS
  chunk = 80
  assert b_per_w % chunk == 0
  nbuf = 2
  n_groups = (b_per_w // chunk) // nbuf

  mesh = plsc.VectorSubcoreMesh(
      core_axis_name="c", subcore_axis_name="s",
      num_cores=_NUM_CORES, num_subcores=_NUM_SUBCORES)

  @functools.partial(
      pl.kernel,
      mesh=mesh,
      out_type=jax.ShapeDtypeStruct((B, D), jnp.float32),
      scratch_types=[
          pltpu.VMEM((b_per_w,), jnp.int32),
          pltpu.VMEM((nbuf, chunk, D), jnp.float32),
          pltpu.SemaphoreType.DMA,
      ],
  )
  def gather_kernel(table_hbm, idx_hbm, out_hbm, idx_v, rows_v, gsem):
    wid = lax.axis_index("s") * _NUM_CORES + lax.axis_index("c")
    base = wid * b_per_w
    pltpu.sync_copy(idx_hbm.at[pl.ds(base, b_per_w)], idx_v)

    def body(grp, carry):
      copies = []
      for b in range(nbuf):
        off = (grp * nbuf + b) * chunk
        copies.append(
            pltpu.async_copy(
                table_hbm.at[idx_v.at[pl.ds(off, chunk)]],
                rows_v.at[b], gsem))
      for b in range(nbuf):
        copies[b].wait()
      for b in range(nbuf):
        off = (grp * nbuf + b) * chunk
        pltpu.sync_copy(rows_v.at[b], out_hbm.at[pl.ds(base + off, chunk)])
      return carry

    lax.fori_loop(0, n_groups, body, 0)

  return gather_kernel


def kernel(word_inputs, word_seq_lengths, char_inputs, char_seq_lengths,
           char_seq_recover, W):
  B, S = word_inputs.shape
  V, D = W.shape
  idx = word_inputs.reshape(B * S).astype(jnp.int32)
  out = _make_sc_gather(V, D, B * S)(W, idx)
  return out.reshape(B, S, D)


# speedup vs baseline: 1.2432x; 1.0730x over previous
import functools

import jax
import jax.numpy as jnp
from jax import lax
from jax.experimental import pallas as pl
from jax.experimental.pallas import tpu as pltpu
from jax.experimental.pallas import tpu_sc as plsc

_NUM_CORES = 2
_NUM_SUBCORES = 16
_NUM_WORKERS = _NUM_CORES * _NUM_SUBCORES


@functools.lru_cache(maxsize=None)
def _make_sc_gather(V, D, B):
  assert B % _NUM_WORKERS == 0
  b_per_w = B // _NUM_WORKERS
  chunk = 80
  group = 320
  cpg = group // chunk
  n_groups = b_per_w // group
  assert b_per_w % group == 0 and group % chunk == 0

  mesh = plsc.VectorSubcoreMesh(
      core_axis_name="c", subcore_axis_name="s",
      num_cores=_NUM_CORES, num_subcores=_NUM_SUBCORES)

  @functools.partial(
      pl.kernel,
      mesh=mesh,
      out_type=jax.ShapeDtypeStruct((B, D), jnp.float32),
      scratch_types=[
          pltpu.VMEM((b_per_w,), jnp.int32),
          pltpu.VMEM((2, group, D), jnp.float32),
          pltpu.SemaphoreType.DMA,
          pltpu.SemaphoreType.DMA,
      ],
  )
  def gather_kernel(table_hbm, idx_hbm, out_hbm, idx_v, rows_v, gsem, wsem):
    wid = lax.axis_index("s") * _NUM_CORES + lax.axis_index("c")
    base = wid * b_per_w
    pltpu.sync_copy(idx_hbm.at[pl.ds(base, b_per_w)], idx_v)

    def start_group(g):
      buf = rows_v.at[g % 2]
      return [
          pltpu.async_copy(
              table_hbm.at[idx_v.at[pl.ds(g * group + c * chunk, chunk)]],
              buf.at[pl.ds(c * chunk, chunk)], gsem)
          for c in range(cpg)
      ]

    gathers = start_group(0)
    writes = [None] * n_groups
    for g in range(n_groups):
      if g + 1 < n_groups:
        if g >= 1:
          writes[g - 1].wait()
        nxt = start_group(g + 1)
      for cp in gathers:
        cp.wait()
      writes[g] = pltpu.async_copy(
          rows_v.at[g % 2], out_hbm.at[pl.ds(base + g * group, group)], wsem)
      if g + 1 < n_groups:
        gathers = nxt
    writes[n_groups - 2].wait()
    writes[n_groups - 1].wait()

  return gather_kernel


def kernel(word_inputs, word_seq_lengths, char_inputs, char_seq_lengths,
           char_seq_recover, W):
  B, S = word_inputs.shape
  V, D = W.shape
  idx = word_inputs.reshape(B * S).astype(jnp.int32)
  out = _make_sc_gather(V, D, B * S)(W, idx)
  return out.reshape(B, S, D)
